# baseline (speedup 1.0000x reference)
import jax
import jax.numpy as jnp
from jax import lax
from jax.experimental import pallas as pl
from jax.experimental.pallas import tpu as pltpu
from jax.experimental.pallas import tpu_sc as plsc

B = 16384
DIM = 32
NC, NS = 2, 16
NW = NC * NS
BPW = B // NW


def _gather_body(uid_hbm, iid_hbm, ut_hbm, it_hbm, ou_hbm, oi_hbm,
                 idx_u, idx_i, rows_u, rows_i, sem_u, sem_i):
  wid = lax.axis_index("c") * NS + lax.axis_index("s")
  base = wid * BPW
  pltpu.sync_copy(uid_hbm.at[pl.ds(base, BPW)], idx_u)
  pltpu.sync_copy(iid_hbm.at[pl.ds(base, BPW)], idx_i)
  cu = pltpu.async_copy(ut_hbm.at[idx_u], rows_u, sem_u)
  ci = pltpu.async_copy(it_hbm.at[idx_i], rows_i, sem_i)
  cu.wait()
  pltpu.sync_copy(rows_u, ou_hbm.at[pl.ds(base, BPW)])
  ci.wait()
  pltpu.sync_copy(rows_i, oi_hbm.at[pl.ds(base, BPW)])


def _gather(user_id, item_id, ut, it):
  mesh = plsc.VectorSubcoreMesh(core_axis_name="c", subcore_axis_name="s",
                                num_cores=NC, num_subcores=NS)
  k = pl.kernel(
      _gather_body,
      out_type=[jax.ShapeDtypeStruct((B, DIM), jnp.float32),
                jax.ShapeDtypeStruct((B, DIM), jnp.float32)],
      mesh=mesh,
      scratch_types=[
          pltpu.VMEM((BPW,), jnp.int32),
          pltpu.VMEM((BPW,), jnp.int32),
          pltpu.VMEM((BPW, DIM), jnp.float32),
          pltpu.VMEM((BPW, DIM), jnp.float32),
          pltpu.SemaphoreType.DMA,
          pltpu.SemaphoreType.DMA,
      ],
      name="ncf_sc_gather",
      compiler_params=pltpu.CompilerParams(use_tc_tiling_on_sc=False),
  )
  return k(user_id, item_id, ut, it)


_SMALL_SIZES = (50, 20, 100, 2)


def _demo_body(gid, lid, aid, gnd,
               genre_t, lang_t, age_t, gender_t,
               w1g, w1l, w1a, w1n, b1, out):
  bm = gid.shape[0]
  acc = jnp.broadcast_to(b1[...], (bm, 64))
  ids = (gid, lid, aid, gnd)
  tabs = (genre_t, lang_t, age_t, gender_t)
  ws = (w1g, w1l, w1a, w1n)
  for t in range(4):
    ncat = _SMALL_SIZES[t]
    proj = jnp.dot(tabs[t][...], ws[t][...],
                   preferred_element_type=jnp.float32)
    cats = jax.lax.broadcasted_iota(jnp.int32, (bm, ncat), 1)
    onehot = (ids[t][...] == cats).astype(jnp.float32)
    acc = acc + jnp.dot(onehot, proj, preferred_element_type=jnp.float32)
  out[...] = acc


def _demo(gid, lid, aid, gnd, genre_emb, lang_emb, age_emb, gender_emb,
          w1ts, b1):
  bm = 2048
  grid = (B // bm,)
  id_spec = pl.BlockSpec((bm, 1), lambda i: (i, 0))
  full = lambda shape: pl.BlockSpec(shape, lambda i: (0, 0))
  in_specs = ([id_spec] * 4
              + [full((n, DIM)) for n in _SMALL_SIZES]
              + [full((DIM, 64))] * 4 + [full((1, 64))])
  return pl.pallas_call(
      _demo_body,
      grid=grid,
      in_specs=in_specs,
      out_specs=pl.BlockSpec((bm, 64), lambda i: (i, 0)),
      out_shape=jax.ShapeDtypeStruct((B, 64), jnp.float32),
  )(gid.reshape(B, 1), lid.reshape(B, 1), aid.reshape(B, 1),
    gnd.reshape(B, 1), genre_emb, lang_emb, age_emb, gender_emb,
    *w1ts[2:], b1)


def _mlp_body(xu, xi, acc0, w1u, w1i, w2t, b2, w3t, b3, out):
  acc = acc0[...]
  acc = acc + jnp.dot(xu[...], w1u[...], preferred_element_type=jnp.float32)
  acc = acc + jnp.dot(xi[...], w1i[...], preferred_element_type=jnp.float32)
  h1 = jnp.maximum(acc, 0.0)
  h2 = jnp.maximum(jnp.dot(h1, w2t[...], preferred_element_type=jnp.float32)
                   + b2[...], 0.0)
  z = jnp.dot(h2, w3t[...], preferred_element_type=jnp.float32) + b3[...]
  out[...] = 1.0 / (1.0 + jnp.exp(-z))


def _mlp(xu, xi, acc0, w1ts, w2t, b2, w3t, b3):
  bm = 2048
  grid = (B // bm,)
  x_spec = pl.BlockSpec((bm, DIM), lambda i: (i, 0))
  full = lambda shape: pl.BlockSpec(shape, lambda i: (0, 0))
  in_specs = [x_spec, x_spec, pl.BlockSpec((bm, 64), lambda i: (i, 0)),
              full((DIM, 64)), full((DIM, 64)),
              full((64, 32)), full((1, 32)), full((32, 1)), full((1, 1))]
  return pl.pallas_call(
      _mlp_body,
      grid=grid,
      in_specs=in_specs,
      out_specs=pl.BlockSpec((bm, 1), lambda i: (i, 0)),
      out_shape=jax.ShapeDtypeStruct((B, 1), jnp.float32),
  )(xu, xi, acc0, w1ts[0], w1ts[1], w2t, b2, w3t, b3)


def kernel(user_id, item_id, genre_id, language_id, age, gender,
           user_emb, item_emb, genre_emb, lang_emb, age_emb, gender_emb,
           W1, b1, W2, b2, W3, b3):
  xu, xi = _gather(user_id.astype(jnp.int32), item_id.astype(jnp.int32),
                   user_emb, item_emb)

  w1t = W1.T
  w1ts = [w1t[DIM * t:DIM * (t + 1)] for t in range(6)]
  acc0 = _demo(genre_id.astype(jnp.int32), language_id.astype(jnp.int32),
               age.astype(jnp.int32), gender.astype(jnp.int32),
               genre_emb, lang_emb, age_emb, gender_emb,
               w1ts, b1.reshape(1, 64))
  return _mlp(xu, xi, acc0, w1ts, W2.T, b2.reshape(1, 32),
              W3.T, b3.reshape(1, 1))
